# baseline (speedup 1.0000x reference)
import functools

import jax
import jax.numpy as jnp
from jax import lax
from jax.experimental import pallas as pl
from jax.experimental.pallas import tpu as pltpu
from jax.experimental.pallas import tpu_sc as plsc

DIM = 64
HDIM = DIM // 2
MAXF = 432000
SECONDS = 7200
B_TOTAL = 1024 * 200
L = 16
NC, NS = 2, 16
NW = NC * NS
SUB = 128
ROWS_PER_W = B_TOTAL // NW
NSUB = ROWS_PER_W // SUB

_INV60 = 1.0 / 60.0
_HI = jnp.int32(-65536)

_mesh = plsc.VectorSubcoreMesh(core_axis_name="c", subcore_axis_name="s")


@functools.partial(
    pl.kernel,
    mesh=_mesh,
    compiler_params=pltpu.CompilerParams(use_tc_tiling_on_sc=False),
    out_type=jax.ShapeDtypeStruct((B_TOTAL, HDIM), jnp.int32),
    scratch_types=[
        pltpu.VMEM((NSUB, SUB), jnp.int32),
        pltpu.VMEM((NSUB, SUB), jnp.int32),
        pltpu.VMEM((2, SUB, HDIM), jnp.int32),
        pltpu.VMEM((2, SUB, HDIM), jnp.int32),
        pltpu.VMEM((2, SUB, HDIM), jnp.int32),
        pltpu.SemaphoreType.DMA,
        pltpu.SemaphoreType.DMA,
    ],
)
def _encode(idx_hbm, ftab, stab, out_hbm,
            idx_v, sidx_v, fbuf, sbuf, obuf, sem0, sem1):
    wid = lax.axis_index("s") * NC + lax.axis_index("c")
    row0 = wid * ROWS_PER_W
    sems = (sem0, sem1)

    pltpu.sync_copy(idx_hbm.at[wid], idx_v)

    def derive(j, carry):
        for k in range(SUB // L):
            s = pl.ds(k * L, L)
            f = idx_v[j, s].astype(jnp.float32)
            sidx_v[j, s] = (f * _INV60).astype(jnp.int32)
        return carry

    lax.fori_loop(0, NSUB, derive, 0)

    def gathers(j, b):
        return [
            pltpu.make_async_copy(ftab.at[idx_v.at[j]], fbuf.at[b], sems[b]),
            pltpu.make_async_copy(stab.at[sidx_v.at[j]], sbuf.at[b], sems[b]),
        ]

    def fire(j, b):
        for cp in gathers(j, b):
            cp.start()

    def bc_f(x):
        return lax.bitcast_convert_type(x, jnp.float32)

    def bc_i(x):
        return lax.bitcast_convert_type(x, jnp.int32)

    fire(0, 0)

    def pair(i, carry):
        j0 = 2 * i
        for b in range(2):
            j = j0 + b

            @pl.when(j < NSUB - 1)
            def _():
                fire(j + 1, 1 - b)

            for cp in gathers(j, b):
                cp.wait()

            def add_rows(r2, c2):
                for rr in range(2):
                    r = r2 * 2 + rr
                    for g in range(HDIM // L):
                        s = pl.ds(g * L, L)
                        cv = fbuf[b, r, s]
                        sv = sbuf[b, r, s]
                        ev = bc_f(cv << 16) + bc_f(sv << 16)
                        ov = (bc_f(lax.bitwise_and(cv, _HI))
                              + bc_f(lax.bitwise_and(sv, _HI)))
                        rnd = jnp.int32(0x8000)
                        obuf[b, r, s] = lax.bitwise_or(
                            lax.shift_right_logical(bc_i(ev) + rnd, 16),
                            lax.bitwise_and(bc_i(ov) + rnd, _HI))
                return c2

            lax.fori_loop(0, SUB // 2, add_rows, 0)
            pltpu.sync_copy(obuf.at[b], out_hbm.at[pl.ds(row0 + j * SUB, SUB)])
        return carry

    lax.fori_loop(0, NSUB // 2, pair, 0)


def kernel(frame_indices, frame_table, second_table, minute_table, pe):
    bsz, seq = frame_indices.shape
    idx = frame_indices.astype(jnp.int32).reshape(NW, NSUB, SUB)

    def pack_bf16_pairs(tab):
        ti = lax.bitcast_convert_type(tab, jnp.int32)
        rnd = jnp.int32(0x8000)
        even = lax.shift_right_logical(ti[:, 0::2] + rnd, 16)
        odd = lax.bitwise_and(ti[:, 1::2] + rnd, _HI)
        return lax.bitwise_or(even, odd)

    comb_i = pack_bf16_pairs(frame_table + pe)
    small_i = pack_bf16_pairs(second_table
                              + jnp.repeat(minute_table, 60, axis=0))
    out_i = _encode(idx, comb_i, small_i)
    ef = lax.bitcast_convert_type(out_i << 16, jnp.float32)
    of = lax.bitcast_convert_type(lax.bitwise_and(out_i, _HI), jnp.float32)
    out = jnp.stack([ef, of], axis=-1).reshape(B_TOTAL, DIM)
    return out.reshape(bsz, seq, DIM)


# speedup vs baseline: 4.9296x; 4.9296x over previous
import functools

import jax
import jax.numpy as jnp
from jax import lax
from jax.experimental import pallas as pl
from jax.experimental.pallas import tpu as pltpu
from jax.experimental.pallas import tpu_sc as plsc

DIM = 64
HDIM = DIM // 2
MAXF = 432000
SECONDS = 7200
B_TOTAL = 1024 * 200
L = 16
NC, NS = 2, 16
NW = NC * NS
SUB = 128
ROWS_PER_W = B_TOTAL // NW
NSUB = ROWS_PER_W // SUB

_INV60 = 1.0 / 60.0
_HI = jnp.int32(-65536)

_mesh = plsc.VectorSubcoreMesh(core_axis_name="c", subcore_axis_name="s")


@functools.partial(
    pl.kernel,
    mesh=_mesh,
    compiler_params=pltpu.CompilerParams(use_tc_tiling_on_sc=False),
    out_type=jax.ShapeDtypeStruct((B_TOTAL, HDIM), jnp.int32),
    scratch_types=[
        pltpu.VMEM((NSUB, SUB), jnp.int32),
        pltpu.VMEM((NSUB, SUB), jnp.int32),
        pltpu.VMEM((2, SUB, HDIM), jnp.int32),
        pltpu.VMEM((2, SUB, HDIM), jnp.int32),
        pltpu.VMEM((2, SUB, HDIM), jnp.int32),
        pltpu.SemaphoreType.DMA,
        pltpu.SemaphoreType.DMA,
    ],
)
def _encode(idx_hbm, ftab, stab, out_hbm,
            idx_v, sidx_v, fbuf, sbuf, obuf, sem0, sem1):
    wid = lax.axis_index("s") * NC + lax.axis_index("c")
    row0 = wid * ROWS_PER_W
    sems = (sem0, sem1)

    pltpu.sync_copy(idx_hbm.at[wid], idx_v)

    def derive(j, carry):
        for k in range(SUB // L):
            s = pl.ds(k * L, L)
            f = idx_v[j, s].astype(jnp.float32)
            sidx_v[j, s] = (f * _INV60).astype(jnp.int32)
        return carry

    lax.fori_loop(0, NSUB, derive, 0)

    def gathers(j, b):
        return [
            pltpu.make_async_copy(ftab.at[idx_v.at[j]], fbuf.at[b], sems[b]),
            pltpu.make_async_copy(stab.at[sidx_v.at[j]], sbuf.at[b], sems[b]),
        ]

    def fire(j, b):
        for cp in gathers(j, b):
            cp.start()

    def bc_f(x):
        return lax.bitcast_convert_type(x, jnp.float32)

    def bc_i(x):
        return lax.bitcast_convert_type(x, jnp.int32)

    fire(0, 0)

    def pair(i, carry):
        j0 = 2 * i
        for b in range(2):
            j = j0 + b

            @pl.when(j < NSUB - 1)
            def _():
                fire(j + 1, 1 - b)

            for cp in gathers(j, b):
                cp.wait()

            def add_rows(r2, c2):
                for rr in range(2):
                    r = r2 * 2 + rr
                    for g in range(HDIM // L):
                        s = pl.ds(g * L, L)
                        cv = fbuf[b, r, s]
                        sv = sbuf[b, r, s]
                        ev = bc_f(cv << 16) + bc_f(sv << 16)
                        ov = (bc_f(lax.bitwise_and(cv, _HI))
                              + bc_f(lax.bitwise_and(sv, _HI)))
                        rnd = jnp.int32(0x8000)
                        obuf[b, r, s] = lax.bitwise_or(
                            lax.shift_right_logical(bc_i(ev) + rnd, 16),
                            lax.bitwise_and(bc_i(ov) + rnd, _HI))
                return c2

            lax.fori_loop(0, SUB // 2, add_rows, 0)
            pltpu.sync_copy(obuf.at[b], out_hbm.at[pl.ds(row0 + j * SUB, SUB)])
        return carry

    lax.fori_loop(0, NSUB // 2, pair, 0)


def kernel(frame_indices, frame_table, second_table, minute_table, pe):
    bsz, seq = frame_indices.shape
    idx = frame_indices.astype(jnp.int32).reshape(NW, NSUB, SUB)

    def pack_bf16_halves(tab):
        ti = lax.bitcast_convert_type(tab, jnp.int32)
        rnd = jnp.int32(0x8000)
        lo = lax.shift_right_logical(ti[:, :HDIM] + rnd, 16)
        hi = lax.bitwise_and(ti[:, HDIM:] + rnd, _HI)
        return lax.bitwise_or(lo, hi)

    comb_i = pack_bf16_halves(frame_table + pe)
    small_i = pack_bf16_halves(second_table
                               + jnp.repeat(minute_table, 60, axis=0))
    out_i = _encode(idx, comb_i, small_i)
    ef = lax.bitcast_convert_type(out_i << 16, jnp.float32)
    of = lax.bitcast_convert_type(lax.bitwise_and(out_i, _HI), jnp.float32)
    out = jnp.concatenate([ef, of], axis=1)
    return out.reshape(bsz, seq, DIM)


# speedup vs baseline: 6.4088x; 1.3001x over previous
import functools

import jax
import jax.numpy as jnp
from jax import lax
from jax.experimental import pallas as pl
from jax.experimental.pallas import tpu as pltpu
from jax.experimental.pallas import tpu_sc as plsc

DIM = 64
HDIM = DIM // 2
MAXF = 432000
SECONDS = 7200
B_TOTAL = 1024 * 200
L = 16
NC, NS = 2, 16
NW = NC * NS
SUB = 128
ROWS_PER_W = B_TOTAL // NW
NSUB = ROWS_PER_W // SUB

_INV60 = 1.0 / 60.0
_HI = jnp.int32(-65536)

_mesh = plsc.VectorSubcoreMesh(core_axis_name="c", subcore_axis_name="s")


@functools.partial(
    pl.kernel,
    mesh=_mesh,
    compiler_params=pltpu.CompilerParams(use_tc_tiling_on_sc=False),
    out_type=jax.ShapeDtypeStruct((B_TOTAL, HDIM), jnp.int32),
    scratch_types=[
        pltpu.VMEM((NSUB, SUB), jnp.int32),
        pltpu.VMEM((NSUB, SUB), jnp.int32),
        pltpu.VMEM((2, SUB, HDIM), jnp.int32),
        pltpu.VMEM((2, SUB, HDIM), jnp.int32),
        pltpu.VMEM((2, SUB, HDIM), jnp.int32),
        pltpu.SemaphoreType.DMA,
        pltpu.SemaphoreType.DMA,
    ],
)
def _encode(idx_hbm, ftab, stab, out_hbm,
            idx_v, sidx_v, fbuf, sbuf, obuf, sem0, sem1):
    wid = lax.axis_index("s") * NC + lax.axis_index("c")
    row0 = wid * ROWS_PER_W
    sems = (sem0, sem1)

    pltpu.sync_copy(idx_hbm.at[wid], idx_v)

    def derive(j, carry):
        for k in range(SUB // L):
            s = pl.ds(k * L, L)
            f = idx_v[j, s].astype(jnp.float32)
            sidx_v[j, s] = (f * _INV60).astype(jnp.int32)
        return carry

    lax.fori_loop(0, NSUB, derive, 0)

    def gathers(j, b):
        return [
            pltpu.make_async_copy(ftab.at[idx_v.at[j]], fbuf.at[b], sems[b]),
            pltpu.make_async_copy(stab.at[sidx_v.at[j]], sbuf.at[b], sems[b]),
        ]

    def fire(j, b):
        for cp in gathers(j, b):
            cp.start()

    def bc_f(x):
        return lax.bitcast_convert_type(x, jnp.float32)

    def bc_i(x):
        return lax.bitcast_convert_type(x, jnp.int32)

    fire(0, 0)

    def pair(i, carry):
        j0 = 2 * i
        for b in range(2):
            j = j0 + b

            @pl.when(j < NSUB - 1)
            def _():
                fire(j + 1, 1 - b)

            for cp in gathers(j, b):
                cp.wait()

            def add_rows(r2, c2):
                for rr in range(2):
                    r = r2 * 2 + rr
                    for g in range(HDIM // L):
                        s = pl.ds(g * L, L)
                        cv = fbuf[b, r, s]
                        sv = sbuf[b, r, s]
                        ev = bc_f(cv << 16) + bc_f(sv << 16)
                        ov = (bc_f(lax.bitwise_and(cv, _HI))
                              + bc_f(lax.bitwise_and(sv, _HI)))
                        rnd = jnp.int32(0x8000)
                        obuf[b, r, s] = lax.bitwise_or(
                            lax.shift_right_logical(bc_i(ev) + rnd, 16),
                            lax.bitwise_and(bc_i(ov) + rnd, _HI))
                return c2

            lax.fori_loop(0, SUB // 2, add_rows, 0)
            pltpu.sync_copy(obuf.at[b], out_hbm.at[pl.ds(row0 + j * SUB, SUB)])
        return carry

    lax.fori_loop(0, NSUB // 2, pair, 0)


def kernel(frame_indices, frame_table, second_table, minute_table, pe):
    bsz, seq = frame_indices.shape
    idx = frame_indices.astype(jnp.int32).reshape(NW, NSUB, SUB)

    def pack_bf16_halves(tab):
        tb = tab.astype(jnp.bfloat16)
        lo = lax.bitcast_convert_type(tb[:, :HDIM], jnp.uint16).astype(jnp.uint32)
        hi = lax.bitcast_convert_type(tb[:, HDIM:], jnp.uint16).astype(jnp.uint32)
        return lax.bitcast_convert_type(lo | (hi << 16), jnp.int32)

    comb_i, small_i = lax.optimization_barrier((
        pack_bf16_halves(frame_table + pe),
        pack_bf16_halves(second_table
                         + jnp.repeat(minute_table, 60, axis=0)),
    ))
    out_i = _encode(idx, comb_i, small_i)
    ef = lax.bitcast_convert_type(out_i << 16, jnp.float32)
    of = lax.bitcast_convert_type(lax.bitwise_and(out_i, _HI), jnp.float32)
    out = jnp.concatenate([ef, of], axis=1)
    return out.reshape(bsz, seq, DIM)


# speedup vs baseline: 9.1451x; 1.4270x over previous
import functools

import jax
import jax.numpy as jnp
from jax import lax
from jax.experimental import pallas as pl
from jax.experimental.pallas import tpu as pltpu
from jax.experimental.pallas import tpu_sc as plsc

DIM = 64
MAXF = 432000
B_TOTAL = 1024 * 200
L = 16
NC, NS = 2, 16
NW = NC * NS
SUB = 128
ROWS_PER_W = B_TOTAL // NW
NSUB = ROWS_PER_W // SUB

_INV60 = 1.0 / 60.0

_mesh = plsc.VectorSubcoreMesh(core_axis_name="c", subcore_axis_name="s")


@functools.partial(
    pl.kernel,
    mesh=_mesh,
    compiler_params=pltpu.CompilerParams(use_tc_tiling_on_sc=False),
    out_type=jax.ShapeDtypeStruct((B_TOTAL, DIM), jnp.float32),
    scratch_types=[
        pltpu.VMEM((NSUB, SUB), jnp.int32),
        pltpu.VMEM((NSUB, SUB), jnp.int32),
        pltpu.VMEM((2, SUB, DIM), jnp.float32),
        pltpu.VMEM((2, SUB, DIM), jnp.float32),
        pltpu.SemaphoreType.DMA,
        pltpu.SemaphoreType.DMA,
    ],
)
def _encode(idx_hbm, ftab, stab, out_hbm,
            idx_v, sidx_v, fbuf, sbuf, sem0, sem1):
    wid = lax.axis_index("s") * NC + lax.axis_index("c")
    row0 = wid * ROWS_PER_W
    sems = (sem0, sem1)

    pltpu.sync_copy(idx_hbm.at[wid], idx_v)

    def derive(j, carry):
        for k in range(SUB // L):
            s = pl.ds(k * L, L)
            f = idx_v[j, s].astype(jnp.float32)
            sidx_v[j, s] = (f * _INV60).astype(jnp.int32)
        return carry

    lax.fori_loop(0, NSUB, derive, 0)

    def gathers(j, b):
        return [
            pltpu.make_async_copy(ftab.at[idx_v.at[j]], fbuf.at[b], sems[b]),
            pltpu.make_async_copy(stab.at[sidx_v.at[j]], sbuf.at[b], sems[b]),
        ]

    def fire(j, b):
        for cp in gathers(j, b):
            cp.start()

    fire(0, 0)

    def pair(i, carry):
        j0 = 2 * i
        for b in range(2):
            j = j0 + b

            @pl.when(j < NSUB - 1)
            def _():
                fire(j + 1, 1 - b)

            for cp in gathers(j, b):
                cp.wait()

            def add_rows(r4, c2):
                for rr in range(4):
                    r = r4 * 4 + rr
                    for q in range(DIM // L):
                        s = pl.ds(q * L, L)
                        fbuf[b, r, s] = fbuf[b, r, s] + sbuf[b, r, s]
                return c2

            lax.fori_loop(0, SUB // 4, add_rows, 0)
            pltpu.sync_copy(fbuf.at[b], out_hbm.at[pl.ds(row0 + j * SUB, SUB)])
        return carry

    lax.fori_loop(0, NSUB // 2, pair, 0)


def kernel(frame_indices, frame_table, second_table, minute_table, pe):
    bsz, seq = frame_indices.shape
    idx = frame_indices.astype(jnp.int32).reshape(NW, NSUB, SUB)
    comb = frame_table + pe
    small = second_table + jnp.repeat(minute_table, 60, axis=0)
    out = _encode(idx, comb, small)
    return out.reshape(bsz, seq, DIM)
